# baseline (speedup 1.0000x reference)
import functools
import math

import jax
import jax.numpy as jnp
import numpy as np
from jax import lax
from jax.experimental import pallas as pl
from jax.experimental.pallas import tpu as pltpu
from jax.experimental.pallas import tpu_sc as plsc

NUM_EMBEDDINGS = 1000000
D = 64
SCALE = math.sqrt(D)

NW = 32
CHUNK = 100
N_CHUNKS = 64
PER_W = CHUNK * N_CHUNKS
B = NW * PER_W
NBUF = 8
N_ROUNDS = N_CHUNKS // NBUF

GRP = 1024
FMT_GRID = (NUM_EMBEDDINGS + 2 * GRP - 1) // (2 * GRP)
NPAIR = FMT_GRID * GRP


def _pe_block() -> np.ndarray:
    pos = np.arange(0, 50).reshape(-1, 1).astype(np.float32)
    even = np.arange(0, D, 2).astype(np.float32)
    power = -even * math.log(10000.0) / D
    pe = np.zeros((50, D), dtype=np.float32)
    pe[:, 0::2] = np.sin(pos * np.exp(power))
    pe[:, 1::2] = np.cos(pos * np.exp(power))
    return np.tile(pe, (CHUNK // 50, 1))


_PE = _pe_block()


def _fmt_body(in_ref, out_ref):
    x = in_ref[...]
    eye = (
        jax.lax.broadcasted_iota(jnp.int32, (D, D), 0)
        == jax.lax.broadcasted_iota(jnp.int32, (D, D), 1)
    ).astype(jnp.float32)
    dims = (((0,), (0,)), ((), ()))
    out_ref[:, 0:D] = jax.lax.dot_general(
        x[:, 0:GRP], eye, dims, preferred_element_type=jnp.float32
    )
    out_ref[:, D : 2 * D] = jax.lax.dot_general(
        x[:, GRP : 2 * GRP], eye, dims, preferred_element_type=jnp.float32
    )


_fmt = pl.pallas_call(
    _fmt_body,
    grid=(FMT_GRID,),
    in_specs=[pl.BlockSpec((D, 2 * GRP), lambda g: (0, g))],
    out_specs=pl.BlockSpec((GRP, 2 * D), lambda g: (g, 0)),
    out_shape=jax.ShapeDtypeStruct((NPAIR, 2 * D), jnp.float32),
)

_mesh = plsc.VectorSubcoreMesh(core_axis_name="c", subcore_axis_name="s")


@functools.partial(
    pl.kernel,
    out_type=jax.ShapeDtypeStruct((B // CHUNK, CHUNK, D), jnp.float32),
    mesh=_mesh,
    compiler_params=pltpu.CompilerParams(use_tc_tiling_on_sc=False),
    scratch_types=[
        pltpu.VMEM((N_CHUNKS, CHUNK), jnp.int32),
        pltpu.VMEM((CHUNK, D), jnp.float32),
        pltpu.VMEM((NBUF, CHUNK, D), jnp.float32),
    ]
    + [pltpu.SemaphoreType.DMA] * (2 * NBUF),
)
def _emb_sc(idx_hbm, lut_hbm, pe_hbm, out_hbm, idx_v, pe_v, rows_v, *sems):
    gsems = sems[:NBUF]
    wsems = sems[NBUF:]
    wid = lax.axis_index("s") * 2 + lax.axis_index("c")
    pltpu.sync_copy(idx_hbm.at[wid], idx_v)
    pltpu.sync_copy(pe_hbm, pe_v)

    for b in range(NBUF):
        pltpu.async_copy(lut_hbm.at[idx_v.at[b]], rows_v.at[b], gsems[b])

    def round_body(r, carry):
        for b in range(NBUF):
            j = r * NBUF + b
            pltpu.make_async_copy(
                lut_hbm.at[idx_v.at[j]], rows_v.at[b], gsems[b]
            ).wait()

            def row_body(rr, c2):
                for cc in range(D // 16):
                    sl = pl.ds(cc * 16, 16)
                    rows_v[b, rr, sl] = rows_v[b, rr, sl] * SCALE + pe_v[rr, sl]
                return c2

            lax.fori_loop(0, CHUNK, row_body, 0)

            out_slot = out_hbm.at[wid * N_CHUNKS + j]
            pltpu.async_copy(rows_v.at[b], out_slot, wsems[b])

            @pl.when(r < N_ROUNDS - 1)
            def _():
                pltpu.make_async_copy(rows_v.at[b], out_slot, wsems[b]).wait()
                pltpu.async_copy(
                    lut_hbm.at[idx_v.at[j + NBUF]], rows_v.at[b], gsems[b]
                )

        return carry

    lax.fori_loop(0, N_ROUNDS, round_body, 0)

    for b in range(NBUF):
        j = (N_ROUNDS - 1) * NBUF + b
        pltpu.make_async_copy(
            rows_v.at[b], out_hbm.at[wid * N_CHUNKS + j], wsems[b]
        ).wait()


def kernel(x, lut):
    n_seq, seq_len = x.shape
    r = x.astype(jnp.int32)
    idx = (
        jax.lax.shift_left(jax.lax.shift_right_logical(r, 11), 11)
        | jax.lax.shift_left(jax.lax.bitwise_and(r, GRP - 1), 1)
        | jax.lax.bitwise_and(jax.lax.shift_right_logical(r, 10), 1)
    ).reshape(NW, N_CHUNKS, CHUNK)
    lut_f = _fmt(lut.T).reshape(2 * NPAIR, D)
    out = _emb_sc(idx, lut_f, jnp.asarray(_PE))
    return out.reshape(n_seq, seq_len, D)


# speedup vs baseline: 1.2221x; 1.2221x over previous
import functools
import math

import jax
import jax.numpy as jnp
import numpy as np
from jax import lax
from jax.experimental import pallas as pl
from jax.experimental.pallas import tpu as pltpu
from jax.experimental.pallas import tpu_sc as plsc

NUM_EMBEDDINGS = 1000000
D = 64
SCALE = math.sqrt(D)

NW = 32
CHUNK = 100
N_CHUNKS = 64
PER_W = CHUNK * N_CHUNKS
B = NW * PER_W
NBUF = 8
N_ROUNDS = N_CHUNKS // NBUF

TBLK = 4096
FMT_GRID = (NUM_EMBEDDINGS + TBLK - 1) // TBLK


def _pe_block() -> np.ndarray:
    pos = np.arange(0, 50).reshape(-1, 1).astype(np.float32)
    even = np.arange(0, D, 2).astype(np.float32)
    power = -even * math.log(10000.0) / D
    pe = np.zeros((50, D), dtype=np.float32)
    pe[:, 0::2] = np.sin(pos * np.exp(power))
    pe[:, 1::2] = np.cos(pos * np.exp(power))
    return np.tile(pe, (CHUNK // 50, 1))


_PE = _pe_block()


def _fmt_body(in_ref, out_ref):
    out_ref[:, 0:D] = in_ref[...].T


_fmt = pl.pallas_call(
    _fmt_body,
    grid=(FMT_GRID,),
    in_specs=[pl.BlockSpec((D, TBLK), lambda g: (0, g))],
    out_specs=pl.BlockSpec((TBLK, 2 * D), lambda g: (g, 0)),
    out_shape=jax.ShapeDtypeStruct((NUM_EMBEDDINGS, 2 * D), jnp.float32),
)

_mesh = plsc.VectorSubcoreMesh(core_axis_name="c", subcore_axis_name="s")


@functools.partial(
    pl.kernel,
    out_type=jax.ShapeDtypeStruct((B // CHUNK, CHUNK, D), jnp.float32),
    mesh=_mesh,
    compiler_params=pltpu.CompilerParams(use_tc_tiling_on_sc=False),
    scratch_types=[
        pltpu.VMEM((N_CHUNKS, CHUNK), jnp.int32),
        pltpu.VMEM((CHUNK, D), jnp.float32),
        pltpu.VMEM((NBUF, CHUNK, D), jnp.float32),
    ]
    + [pltpu.SemaphoreType.DMA] * (2 * NBUF),
)
def _emb_sc(idx_hbm, lut_hbm, pe_hbm, out_hbm, idx_v, pe_v, rows_v, *sems):
    gsems = sems[:NBUF]
    wsems = sems[NBUF:]
    wid = lax.axis_index("s") * 2 + lax.axis_index("c")
    pltpu.sync_copy(idx_hbm.at[wid], idx_v)
    pltpu.sync_copy(pe_hbm, pe_v)

    for b in range(NBUF):
        pltpu.async_copy(lut_hbm.at[idx_v.at[b]], rows_v.at[b], gsems[b])

    def round_body(r, carry):
        for b in range(NBUF):
            j = r * NBUF + b
            pltpu.make_async_copy(
                lut_hbm.at[idx_v.at[j]], rows_v.at[b], gsems[b]
            ).wait()

            def row_body(rr, c2):
                for cc in range(D // 16):
                    sl = pl.ds(cc * 16, 16)
                    rows_v[b, rr, sl] = rows_v[b, rr, sl] * SCALE + pe_v[rr, sl]
                return c2

            lax.fori_loop(0, CHUNK, row_body, 0)

            out_slot = out_hbm.at[wid * N_CHUNKS + j]
            pltpu.async_copy(rows_v.at[b], out_slot, wsems[b])

            @pl.when(r < N_ROUNDS - 1)
            def _():
                pltpu.make_async_copy(rows_v.at[b], out_slot, wsems[b]).wait()
                pltpu.async_copy(
                    lut_hbm.at[idx_v.at[j + NBUF]], rows_v.at[b], gsems[b]
                )

        return carry

    lax.fori_loop(0, N_ROUNDS, round_body, 0)

    for b in range(NBUF):
        j = (N_ROUNDS - 1) * NBUF + b
        pltpu.make_async_copy(
            rows_v.at[b], out_hbm.at[wid * N_CHUNKS + j], wsems[b]
        ).wait()


def kernel(x, lut):
    n_seq, seq_len = x.shape
    idx = (x.astype(jnp.int32) * 2).reshape(NW, N_CHUNKS, CHUNK)
    lut_f = _fmt(lut.T).reshape(2 * NUM_EMBEDDINGS, D)
    out = _emb_sc(idx, lut_f, jnp.asarray(_PE))
    return out.reshape(n_seq, seq_len, D)
